# Initial kernel scaffold; baseline (speedup 1.0000x reference)
import jax
import jax.numpy as jnp
from jax import lax
from jax.experimental import pallas as pl
from jax.experimental.pallas import tpu as pltpu
from jax.experimental.pallas import tpu_sc as plsc

_N = 10000
_E = 320000
_D = 128
_NT = 10
_NB = 8
_RMAX = 5.0
_EV_TO_KJ_MOL = 96.48533212331
_NROW = _N * _NT

_NC = 2
_NS = 16
_NW = _NC * _NS
_EPT = _E // _NW
_BATCH = 128
_NFULL = _EPT // _BATCH
_TAIL = _EPT - _NFULL * _BATCH
_RPT = _NROW // _NS

_PI = 3.14159265358979
_C_BESSEL = 0.6324555320336759

_COSC = (1.0, -1.0 / 2, 1.0 / 24, -1.0 / 720, 1.0 / 40320,
         -1.0 / 3628800, 1.0 / 479001600)
_SINC = (1.0, -1.0 / 6, 1.0 / 120, -1.0 / 5040, 1.0 / 362880,
         -1.0 / 39916800)


def _f32(x):
    return jnp.float32(x)


def _edge_group(o, pos_v, types_v, snd_v, rcv_v, si_v, c10):
    snd = snd_v[pl.ds(o, 16)]
    rcv = rcv_v[pl.ds(o, 16)]
    cols = [jnp.full((16,), d, jnp.int32) for d in range(3)]
    ps = [plsc.load_gather(pos_v, [snd, cols[d]]) for d in range(3)]
    pr = [plsc.load_gather(pos_v, [rcv, cols[d]]) for d in range(3)]
    t = plsc.load_gather(types_v, [snd])
    sif = [si_v[k, pl.ds(o, 16)].astype(jnp.float32) for k in range(3)]

    d2 = _f32(1e-9)
    for d in range(3):
        v = (pr[d] - ps[d]) * _f32(10.0)
        for k in range(3):
            v = v + sif[k] * c10[k][d]
        d2 = d2 + v * v

    ibits = plsc.bitcast(d2, jnp.int32)
    y = plsc.bitcast(jnp.int32(0x5F3759DF) - lax.shift_right_logical(ibits, 1),
                     jnp.float32)
    for _ in range(3):
        y = y * (_f32(1.5) - _f32(0.5) * d2 * y * y)
    r = d2 * y
    inv_r = y

    x = r * _f32(1.0 / _RMAX)
    xc = jnp.minimum(x, _f32(1.0))

    u = _f32(_PI) * xc - _f32(_PI / 2)
    u2 = u * u
    cp = _f32(_COSC[-1])
    for cc in _COSC[-2::-1]:
        cp = cp * u2 + _f32(cc)
    sp = _f32(_SINC[-1])
    for cc in _SINC[-2::-1]:
        sp = sp * u2 + _f32(cc)
    sp = sp * u
    s1 = cp
    c1 = -sp

    t2 = _f32(2.0) * c1
    sins = [s1, t2 * s1]
    for _ in range(2, _NB):
        sins.append(t2 * sins[-1] - sins[-2])

    x3 = xc * xc * xc
    x6 = x3 * x3
    x7 = x6 * xc
    x8 = x7 * xc
    fc = _f32(1.0) - _f32(28.0) * x6 + _f32(48.0) * x7 - _f32(21.0) * x8
    w = _f32(_C_BESSEL) * fc * inv_r
    w = jnp.where(x < _f32(1.0), w, _f32(0.0))

    rows = rcv * jnp.int32(_NT) + t
    return rows, [s * w for s in sins]


def _sc_body(pos_hbm, types_hbm, edge_hbm, siT_hbm, cell_hbm, zeros_hbm,
             out_hbm,
             pos_v, types_v, snd_v, rcv_v, si_v, stg_v, rowidx_v,
             stg_t, rowidx_t, cell_s, acc_v):
    c_idx = lax.axis_index("c")
    s_idx = lax.axis_index("s")
    wid = s_idx * _NC + c_idx
    base = wid * _EPT

    pltpu.sync_copy(cell_hbm, cell_s)
    pltpu.sync_copy(pos_hbm, pos_v)
    pltpu.sync_copy(types_hbm, types_v)
    pltpu.sync_copy(edge_hbm.at[0, pl.ds(base, _EPT)], snd_v)
    pltpu.sync_copy(edge_hbm.at[1, pl.ds(base, _EPT)], rcv_v)
    for k in range(3):
        pltpu.sync_copy(siT_hbm.at[k, pl.ds(base, _EPT)], si_v.at[k])

    pltpu.sync_copy(zeros_hbm, acc_v.at[pl.ds(s_idx * _RPT, _RPT)])
    plsc.subcore_barrier()

    c10 = [[cell_s[k * 3 + d] * _f32(10.0) for d in range(3)]
           for k in range(3)]

    lane = jnp.arange(16, dtype=jnp.int32)

    def batch_body(b, carry):
        o0 = b * _BATCH
        for g in range(_BATCH // 16):
            rows, rad = _edge_group(o0 + g * 16, pos_v, types_v, snd_v,
                                    rcv_v, si_v, c10)
            rowidx_v[pl.ds(g * 16, 16)] = rows
            for n in range(_NB):
                plsc.store_scatter(
                    stg_v, [lane + jnp.int32(g * 16),
                            jnp.full((16,), n, jnp.int32)], rad[n])
        pltpu.sync_copy(stg_v, acc_v.at[rowidx_v], add=True)
        return carry

    lax.fori_loop(0, _NFULL, batch_body, 0)

    rows, rad = _edge_group(_NFULL * _BATCH, pos_v, types_v, snd_v, rcv_v,
                            si_v, c10)
    rowidx_t[...] = rows
    for n in range(_NB):
        plsc.store_scatter(stg_t, [lane, jnp.full((16,), n, jnp.int32)],
                           rad[n])
    pltpu.sync_copy(stg_t, acc_v.at[rowidx_t], add=True)

    plsc.subcore_barrier()
    pltpu.sync_copy(acc_v.at[pl.ds(s_idx * _RPT, _RPT)],
                    out_hbm.at[c_idx, pl.ds(s_idx * _RPT, _RPT)])


_sc_call = pl.kernel(
    _sc_body,
    out_type=jax.ShapeDtypeStruct((_NC, _NROW, _NB), jnp.float32),
    mesh=plsc.VectorSubcoreMesh(core_axis_name="c", subcore_axis_name="s"),
    scratch_types=[
        pltpu.VMEM((_N, 3), jnp.float32),
        pltpu.VMEM((_N,), jnp.int32),
        pltpu.VMEM((_EPT,), jnp.int32),
        pltpu.VMEM((_EPT,), jnp.int32),
        pltpu.VMEM((3, _EPT), jnp.int32),
        pltpu.VMEM((_BATCH, _NB), jnp.float32),
        pltpu.VMEM((_BATCH,), jnp.int32),
        pltpu.VMEM((16, _NB), jnp.float32),
        pltpu.VMEM((16,), jnp.int32),
        pltpu.SMEM((16,), jnp.float32),
        pltpu.VMEM_SHARED((_NROW, _NB), jnp.float32),
    ],
)


def _tc_body(r2_ref, embed_ref, wmsg_ref, wrad_ref, woutT_ref, out_ref):
    rsum = r2_ref[0] + r2_ref[1]
    hw = jnp.dot(embed_ref[...], wmsg_ref[...],
                 preferred_element_type=jnp.float32)
    wbig = (hw[:, None, :] * wrad_ref[...][None, :, :]).reshape(
        _NT * _NB, _D)
    agg = jnp.dot(rsum, wbig, preferred_element_type=jnp.float32)
    act = agg * jax.nn.sigmoid(agg)
    node_e = act * woutT_ref[...]
    out_ref[0, 0] = jnp.sum(node_e) * jnp.float32(_EV_TO_KJ_MOL)


_tc_call = pl.pallas_call(
    _tc_body,
    out_shape=jax.ShapeDtypeStruct((1, 1), jnp.float32),
)


@jax.jit
def kernel(positions, boxVectors, edge_index, shifts_idx, node_types,
           embed, W_msg, W_rad, W_out):
    cell_flat = jnp.pad(jnp.reshape(boxVectors, (9,)), (0, 7))
    siT = shifts_idx.T
    zeros = jnp.zeros((_RPT, _NB), jnp.float32)
    r2 = _sc_call(positions, node_types, edge_index, siT, cell_flat, zeros)
    r2r = jnp.reshape(r2, (_NC, _N, _NT * _NB))
    out = _tc_call(r2r, embed, W_msg, W_rad, jnp.reshape(W_out, (1, _D)))
    return out[0, 0]


# speedup vs baseline: 4.9044x; 4.9044x over previous
import jax
import jax.numpy as jnp
from jax import lax
from jax.experimental import pallas as pl
from jax.experimental.pallas import tpu as pltpu
from jax.experimental.pallas import tpu_sc as plsc

_N = 10000
_E = 320000
_D = 128
_NT = 10
_NB = 8
_RMAX = 5.0
_EV_TO_KJ_MOL = 96.48533212331

_NC = 2
_NS = 16
_EPT = _E // _NS
_BATCH = 128
_NFULL = _EPT // _BATCH
_TAIL = _EPT - _NFULL * _BATCH

_NBKT = _N * _NT // _NC
_TRASH = _NBKT * _NB
_AWORDS = 409600
_WPT = _AWORDS // _NS

_PI = 3.14159265358979
_C_BESSEL = 0.6324555320336759

_COSC = (1.0, -1.0 / 2, 1.0 / 24, -1.0 / 720, 1.0 / 40320,
         -1.0 / 3628800, 1.0 / 479001600)
_SINC = (1.0, -1.0 / 6, 1.0 / 120, -1.0 / 5040, 1.0 / 362880,
         -1.0 / 39916800)


def _f32(x):
    return jnp.float32(x)


def _edge_group(o, pos_v, types_v, epack_v, sipack_v, c10, c_idx):
    lane = jnp.arange(16, dtype=jnp.int32)
    code = epack_v[pl.ds(o, 16)]
    snd = code & jnp.int32(0xFFFF)
    rcv = lax.shift_right_logical(code, 16)
    e = o + lane
    w27 = plsc.load_gather(sipack_v, [lax.shift_right_logical(e, 1)])
    c27 = lax.shift_right_logical(
        w27, (e & jnp.int32(1)) * jnp.int32(5)) & jnp.int32(31)
    si0 = lax.rem(c27, jnp.int32(3))
    q3 = lax.div(c27, jnp.int32(3))
    si1 = lax.rem(q3, jnp.int32(3))
    si2 = lax.div(q3, jnp.int32(3))
    sif = [si0.astype(jnp.float32), si1.astype(jnp.float32),
           si2.astype(jnp.float32)]

    snd3 = snd * jnp.int32(3)
    rcv3 = rcv * jnp.int32(3)
    ps = [plsc.load_gather(pos_v, [snd3 + jnp.int32(d)]) for d in range(3)]
    pr = [plsc.load_gather(pos_v, [rcv3 + jnp.int32(d)]) for d in range(3)]
    t = plsc.load_gather(types_v, [snd])

    d2 = _f32(1e-9)
    for d in range(3):
        v = (pr[d] - ps[d]) * _f32(10.0)
        for k in range(3):
            v = v + sif[k] * c10[k][d]
        d2 = d2 + v * v

    ibits = plsc.bitcast(d2, jnp.int32)
    y = plsc.bitcast(jnp.int32(0x5F3759DF) - lax.shift_right_logical(ibits, 1),
                     jnp.float32)
    for _ in range(3):
        y = y * (_f32(1.5) - _f32(0.5) * d2 * y * y)
    r = d2 * y
    inv_r = y

    x = r * _f32(1.0 / _RMAX)
    xc = jnp.minimum(x, _f32(1.0))

    u = _f32(_PI) * xc - _f32(_PI / 2)
    u2 = u * u
    cp = _f32(_COSC[-1])
    for cc in _COSC[-2::-1]:
        cp = cp * u2 + _f32(cc)
    sp = _f32(_SINC[-1])
    for cc in _SINC[-2::-1]:
        sp = sp * u2 + _f32(cc)
    sp = sp * u
    s1 = cp
    c1 = -sp

    t2 = _f32(2.0) * c1
    sins = [s1, t2 * s1]
    for _ in range(2, _NB):
        sins.append(t2 * sins[-1] - sins[-2])

    x3 = xc * xc * xc
    x6 = x3 * x3
    x7 = x6 * xc
    x8 = x7 * xc
    fc = _f32(1.0) - _f32(28.0) * x6 + _f32(48.0) * x7 - _f32(21.0) * x8
    w = _f32(_C_BESSEL) * fc * inv_r
    w = jnp.where(x < _f32(1.0), w, _f32(0.0))

    tl = t - jnp.int32(5) * c_idx
    in_half = (tl >= jnp.int32(0)) & (tl < jnp.int32(5))
    basew = jnp.where(in_half,
                      (rcv * jnp.int32(5) + tl) * jnp.int32(_NB),
                      jnp.int32(_TRASH))
    return basew, [s * w for s in sins]


def _sc_body(pos_hbm, types_hbm, epack_hbm, sipack_hbm, cell_hbm, zeros_hbm,
             out_hbm,
             pos_v, types_v, epack_v, sipack_v,
             idx2_v, val2_v, idx2t_v, val2t_v, cell_s, acc_v):
    c_idx = lax.axis_index("c")
    s_idx = lax.axis_index("s")
    base = s_idx * _EPT

    pltpu.sync_copy(cell_hbm, cell_s)
    pltpu.sync_copy(pos_hbm, pos_v)
    pltpu.sync_copy(types_hbm, types_v)
    pltpu.sync_copy(epack_hbm.at[pl.ds(pl.multiple_of(base, 8), _EPT)],
                    epack_v)
    pltpu.sync_copy(
        sipack_hbm.at[pl.ds(pl.multiple_of(base // 2, 8), _EPT // 2)],
        sipack_v)

    pltpu.sync_copy(zeros_hbm,
                    acc_v.at[pl.ds(pl.multiple_of(s_idx * _WPT, 8), _WPT)])
    plsc.subcore_barrier()

    cvec = cell_s[...]
    c10 = [[cvec[k * 3 + d] * _f32(10.0) for d in range(3)]
           for k in range(3)]

    def batch_body(b, carry):
        o0 = b * _BATCH
        for g in range(_BATCH // 16):
            basew, rad = _edge_group(o0 + g * 16, pos_v, types_v, epack_v,
                                     sipack_v, c10, c_idx)
            for n in range(_NB):
                idx2_v[n, pl.ds(g * 16, 16)] = basew + jnp.int32(n)
                val2_v[n, pl.ds(g * 16, 16)] = rad[n]
        for n in range(_NB):
            pltpu.sync_copy(val2_v.at[n], acc_v.at[idx2_v.at[n]], add=True)
        return carry

    lax.fori_loop(0, _NFULL, batch_body, 0)

    for g in range(_TAIL // 16):
        basew, rad = _edge_group(_NFULL * _BATCH + g * 16, pos_v, types_v,
                                 epack_v, sipack_v, c10, c_idx)
        for n in range(_NB):
            p = n * _TAIL + g * 16
            idx2t_v[p // 128, pl.ds(p % 128, 16)] = basew + jnp.int32(n)
            val2t_v[p // 128, pl.ds(p % 128, 16)] = rad[n]
    for n in range(2):
        pltpu.sync_copy(val2t_v.at[n], acc_v.at[idx2t_v.at[n]], add=True)

    plsc.subcore_barrier()
    pltpu.sync_copy(
        acc_v.at[pl.ds(pl.multiple_of(s_idx * _WPT, 8), _WPT)],
        out_hbm.at[pl.ds(
            pl.multiple_of(c_idx * _AWORDS + s_idx * _WPT, 8), _WPT)])


_sc_call = pl.kernel(
    _sc_body,
    out_type=jax.ShapeDtypeStruct((_NC * _AWORDS,), jnp.float32),
    mesh=plsc.VectorSubcoreMesh(core_axis_name="c", subcore_axis_name="s"),
    compiler_params=pltpu.CompilerParams(needs_layout_passes=False),
    scratch_types=[
        pltpu.VMEM((_N * 3,), jnp.float32),
        pltpu.VMEM((_N,), jnp.int32),
        pltpu.VMEM((_EPT,), jnp.int32),
        pltpu.VMEM((_EPT // 2,), jnp.int32),
        pltpu.VMEM((_NB, _BATCH), jnp.int32),
        pltpu.VMEM((_NB, _BATCH), jnp.float32),
        pltpu.VMEM((2, _BATCH), jnp.int32),
        pltpu.VMEM((2, _BATCH), jnp.float32),
        pltpu.VMEM((16,), jnp.float32),
        pltpu.VMEM_SHARED((_AWORDS,), jnp.float32),
    ],
)


def _tc_body(r_ref, embed_ref, wmsg_ref, wrad_ref, woutT_ref, out_ref):
    hw = jnp.dot(embed_ref[...], wmsg_ref[...],
                 preferred_element_type=jnp.float32,
                 precision=lax.Precision.HIGHEST)
    wbig = (hw[:, None, :] * wrad_ref[...][None, :, :]).reshape(
        _NT * _NB, _D)
    agg = jnp.dot(r_ref[...], wbig, preferred_element_type=jnp.float32,
                  precision=lax.Precision.HIGHEST)
    act = agg * jax.nn.sigmoid(agg)
    node_e = act * woutT_ref[...]
    total = jnp.sum(node_e) * jnp.float32(_EV_TO_KJ_MOL)
    out_ref[...] = jnp.reshape(total, (1, 1))


_tc_call = pl.pallas_call(
    _tc_body,
    out_shape=jax.ShapeDtypeStruct((1, 1), jnp.float32),
)


@jax.jit
def kernel(positions, boxVectors, edge_index, shifts_idx, node_types,
           embed, W_msg, W_rad, W_out):
    cell_flat = jnp.pad(jnp.reshape(boxVectors, (9,)), (0, 7))
    epack = edge_index[0] + edge_index[1] * jnp.int32(65536)
    c27 = (shifts_idx[:, 0] + shifts_idx[:, 1] * jnp.int32(3)
           + shifts_idx[:, 2] * jnp.int32(9)).reshape(_E // 2, 2)
    sipack = c27[:, 0] + (c27[:, 1] << 5)
    zeros = jnp.zeros((_WPT,), jnp.float32)
    pos_flat = jnp.reshape(positions, (3 * _N,))
    r2 = _sc_call(pos_flat, node_types, epack, sipack, cell_flat, zeros)
    halves = jnp.reshape(r2, (_NC, _AWORDS))[:, :_NBKT * _NB]
    rfull = jnp.concatenate(
        [jnp.reshape(halves[0], (_N, 5 * _NB)),
         jnp.reshape(halves[1], (_N, 5 * _NB))], axis=1)
    out = _tc_call(rfull, embed, W_msg, W_rad, jnp.reshape(W_out, (1, _D)))
    return out[0, 0]


# speedup vs baseline: 4.9063x; 1.0004x over previous
import jax
import jax.numpy as jnp
from jax import lax
from jax.experimental import pallas as pl
from jax.experimental.pallas import tpu as pltpu
from jax.experimental.pallas import tpu_sc as plsc

_N = 10000
_E = 320000
_D = 128
_NT = 10
_NB = 8
_RMAX = 5.0
_EV_TO_KJ_MOL = 96.48533212331

_NC = 2
_NS = 16
_EPT = _E // _NS
_BATCH = 128
_NFULL = _EPT // _BATCH
_TAIL = _EPT - _NFULL * _BATCH

_NBKT = _N * _NT // _NC
_TRASH = _NBKT * _NB
_AWORDS = 409600
_WPT = _AWORDS // _NS

_PI = 3.14159265358979
_C_BESSEL = 0.6324555320336759

_COSC = (1.0, -1.0 / 2, 1.0 / 24, -1.0 / 720, 1.0 / 40320,
         -1.0 / 3628800, 1.0 / 479001600)
_SINC = (1.0, -1.0 / 6, 1.0 / 120, -1.0 / 5040, 1.0 / 362880,
         -1.0 / 39916800)


def _f32(x):
    return jnp.float32(x)


def _edge_group(o, pos_v, types_v, epack_v, sipack_v, c10, c_idx):
    lane = jnp.arange(16, dtype=jnp.int32)
    code = epack_v[pl.ds(o, 16)]
    snd = code & jnp.int32(0xFFFF)
    rcv = lax.shift_right_logical(code, 16)
    e = o + lane
    w27 = plsc.load_gather(sipack_v, [lax.shift_right_logical(e, 1)])
    c27 = lax.shift_right_logical(
        w27, (e & jnp.int32(1)) * jnp.int32(5)) & jnp.int32(31)
    si0 = lax.rem(c27, jnp.int32(3))
    q3 = lax.div(c27, jnp.int32(3))
    si1 = lax.rem(q3, jnp.int32(3))
    si2 = lax.div(q3, jnp.int32(3))
    sif = [si0.astype(jnp.float32), si1.astype(jnp.float32),
           si2.astype(jnp.float32)]

    snd3 = snd * jnp.int32(3)
    rcv3 = rcv * jnp.int32(3)
    ps = [plsc.load_gather(pos_v, [snd3 + jnp.int32(d)]) for d in range(3)]
    pr = [plsc.load_gather(pos_v, [rcv3 + jnp.int32(d)]) for d in range(3)]
    t = plsc.load_gather(types_v, [snd])

    d2 = _f32(1e-9)
    for d in range(3):
        v = (pr[d] - ps[d]) * _f32(10.0)
        for k in range(3):
            v = v + sif[k] * c10[k][d]
        d2 = d2 + v * v

    ibits = plsc.bitcast(d2, jnp.int32)
    y = plsc.bitcast(jnp.int32(0x5F3759DF) - lax.shift_right_logical(ibits, 1),
                     jnp.float32)
    for _ in range(3):
        y = y * (_f32(1.5) - _f32(0.5) * d2 * y * y)
    r = d2 * y
    inv_r = y

    x = r * _f32(1.0 / _RMAX)
    xc = jnp.minimum(x, _f32(1.0))

    u = _f32(_PI) * xc - _f32(_PI / 2)
    u2 = u * u
    cp = _f32(_COSC[-1])
    for cc in _COSC[-2::-1]:
        cp = cp * u2 + _f32(cc)
    sp = _f32(_SINC[-1])
    for cc in _SINC[-2::-1]:
        sp = sp * u2 + _f32(cc)
    sp = sp * u
    s1 = cp
    c1 = -sp

    t2 = _f32(2.0) * c1
    sins = [s1, t2 * s1]
    for _ in range(2, _NB):
        sins.append(t2 * sins[-1] - sins[-2])

    x3 = xc * xc * xc
    x6 = x3 * x3
    x7 = x6 * xc
    x8 = x7 * xc
    fc = _f32(1.0) - _f32(28.0) * x6 + _f32(48.0) * x7 - _f32(21.0) * x8
    w = _f32(_C_BESSEL) * fc * inv_r
    w = jnp.where(x < _f32(1.0), w, _f32(0.0))

    tl = t - jnp.int32(5) * c_idx
    in_half = (tl >= jnp.int32(0)) & (tl < jnp.int32(5))
    basew = jnp.where(in_half,
                      (rcv * jnp.int32(5) + tl) * jnp.int32(_NB),
                      jnp.int32(_TRASH))
    return basew, [s * w for s in sins]


def _sc_body(pos_hbm, types_hbm, epack_hbm, sipack_hbm, cell_hbm, zeros_hbm,
             out_hbm,
             pos_v, types_v, epack_v, sipack_v,
             idx2_v, val2_v, idx2t_v, val2t_v, cell_s, acc_v, sem):
    c_idx = lax.axis_index("c")
    s_idx = lax.axis_index("s")
    base = s_idx * _EPT

    pltpu.sync_copy(cell_hbm, cell_s)
    pltpu.sync_copy(pos_hbm, pos_v)
    pltpu.sync_copy(types_hbm, types_v)
    pltpu.sync_copy(epack_hbm.at[pl.ds(pl.multiple_of(base, 8), _EPT)],
                    epack_v)
    pltpu.sync_copy(
        sipack_hbm.at[pl.ds(pl.multiple_of(base // 2, 8), _EPT // 2)],
        sipack_v)

    pltpu.sync_copy(zeros_hbm,
                    acc_v.at[pl.ds(pl.multiple_of(s_idx * _WPT, 8), _WPT)])
    plsc.subcore_barrier()

    cvec = cell_s[...]
    c10 = [[cvec[k * 3 + d] * _f32(10.0) for d in range(3)]
           for k in range(3)]

    def batch_body(b, carry):
        o0 = b * _BATCH
        for g in range(_BATCH // 16):
            basew, rad = _edge_group(o0 + g * 16, pos_v, types_v, epack_v,
                                     sipack_v, c10, c_idx)
            for n in range(_NB):
                idx2_v[n, pl.ds(g * 16, 16)] = basew + jnp.int32(n)
                val2_v[n, pl.ds(g * 16, 16)] = rad[n]
        for n in range(_NB):
            pltpu.make_async_copy(val2_v.at[n], acc_v.at[idx2_v.at[n]],
                                  sem).start(add=True)
        for n in range(_NB):
            pltpu.make_async_copy(val2_v.at[n], acc_v.at[idx2_v.at[n]],
                                  sem).wait()
        return carry

    lax.fori_loop(0, _NFULL, batch_body, 0)

    for g in range(_TAIL // 16):
        basew, rad = _edge_group(_NFULL * _BATCH + g * 16, pos_v, types_v,
                                 epack_v, sipack_v, c10, c_idx)
        for n in range(_NB):
            p = n * _TAIL + g * 16
            idx2t_v[p // 128, pl.ds(p % 128, 16)] = basew + jnp.int32(n)
            val2t_v[p // 128, pl.ds(p % 128, 16)] = rad[n]
    for n in range(2):
        pltpu.make_async_copy(val2t_v.at[n], acc_v.at[idx2t_v.at[n]],
                              sem).start(add=True)
    for n in range(2):
        pltpu.make_async_copy(val2t_v.at[n], acc_v.at[idx2t_v.at[n]],
                              sem).wait()

    plsc.subcore_barrier()
    pltpu.sync_copy(
        acc_v.at[pl.ds(pl.multiple_of(s_idx * _WPT, 8), _WPT)],
        out_hbm.at[pl.ds(
            pl.multiple_of(c_idx * _AWORDS + s_idx * _WPT, 8), _WPT)])


_sc_call = pl.kernel(
    _sc_body,
    out_type=jax.ShapeDtypeStruct((_NC * _AWORDS,), jnp.float32),
    mesh=plsc.VectorSubcoreMesh(core_axis_name="c", subcore_axis_name="s"),
    compiler_params=pltpu.CompilerParams(needs_layout_passes=False),
    scratch_types=[
        pltpu.VMEM((_N * 3,), jnp.float32),
        pltpu.VMEM((_N,), jnp.int32),
        pltpu.VMEM((_EPT,), jnp.int32),
        pltpu.VMEM((_EPT // 2,), jnp.int32),
        pltpu.VMEM((_NB, _BATCH), jnp.int32),
        pltpu.VMEM((_NB, _BATCH), jnp.float32),
        pltpu.VMEM((2, _BATCH), jnp.int32),
        pltpu.VMEM((2, _BATCH), jnp.float32),
        pltpu.VMEM((16,), jnp.float32),
        pltpu.VMEM_SHARED((_AWORDS,), jnp.float32),
        pltpu.SemaphoreType.DMA,
    ],
)


def _tc_body(r_ref, embed_ref, wmsg_ref, wrad_ref, woutT_ref, out_ref):
    hw = jnp.dot(embed_ref[...], wmsg_ref[...],
                 preferred_element_type=jnp.float32,
                 precision=lax.Precision.HIGHEST)
    wbig = (hw[:, None, :] * wrad_ref[...][None, :, :]).reshape(
        _NT * _NB, _D)
    agg = jnp.dot(r_ref[...], wbig, preferred_element_type=jnp.float32,
                  precision=lax.Precision.HIGHEST)
    act = agg * jax.nn.sigmoid(agg)
    node_e = act * woutT_ref[...]
    total = jnp.sum(node_e) * jnp.float32(_EV_TO_KJ_MOL)
    out_ref[...] = jnp.reshape(total, (1, 1))


_tc_call = pl.pallas_call(
    _tc_body,
    out_shape=jax.ShapeDtypeStruct((1, 1), jnp.float32),
)


@jax.jit
def kernel(positions, boxVectors, edge_index, shifts_idx, node_types,
           embed, W_msg, W_rad, W_out):
    cell_flat = jnp.pad(jnp.reshape(boxVectors, (9,)), (0, 7))
    epack = edge_index[0] + edge_index[1] * jnp.int32(65536)
    c27 = (shifts_idx[:, 0] + shifts_idx[:, 1] * jnp.int32(3)
           + shifts_idx[:, 2] * jnp.int32(9)).reshape(_E // 2, 2)
    sipack = c27[:, 0] + (c27[:, 1] << 5)
    zeros = jnp.zeros((_WPT,), jnp.float32)
    pos_flat = jnp.reshape(positions, (3 * _N,))
    r2 = _sc_call(pos_flat, node_types, epack, sipack, cell_flat, zeros)
    halves = jnp.reshape(r2, (_NC, _AWORDS))[:, :_NBKT * _NB]
    rfull = jnp.concatenate(
        [jnp.reshape(halves[0], (_N, 5 * _NB)),
         jnp.reshape(halves[1], (_N, 5 * _NB))], axis=1)
    out = _tc_call(rfull, embed, W_msg, W_rad, jnp.reshape(W_out, (1, _D)))
    return out[0, 0]


# speedup vs baseline: 4.9140x; 1.0016x over previous
import jax
import jax.numpy as jnp
from jax import lax
from jax.experimental import pallas as pl
from jax.experimental.pallas import tpu as pltpu
from jax.experimental.pallas import tpu_sc as plsc

_N = 10000
_E = 320000
_D = 128
_NT = 10
_NB = 8
_RMAX = 5.0
_EV_TO_KJ_MOL = 96.48533212331

_NC = 2
_NS = 16
_EPT = _E // _NS
_BATCH = 128
_NFULL = _EPT // _BATCH
_TAIL = _EPT - _NFULL * _BATCH

_NBKT = _N * _NT // _NC
_TRASH = _NBKT * _NB
_AWORDS = 409600
_WPT = _AWORDS // _NS

_PI = 3.14159265358979
_C_BESSEL = 0.6324555320336759

_COSC = (1.0, -1.0 / 2, 1.0 / 24, -1.0 / 720, 1.0 / 40320,
         -1.0 / 3628800, 1.0 / 479001600)
_SINC = (1.0, -1.0 / 6, 1.0 / 120, -1.0 / 5040, 1.0 / 362880,
         -1.0 / 39916800)


def _f32(x):
    return jnp.float32(x)


def _edge_group(o, pos_v, types_v, epack_v, sipack_v, c10, c_idx):
    lane = jnp.arange(16, dtype=jnp.int32)
    code = epack_v[pl.ds(o, 16)]
    snd = code & jnp.int32(0xFFFF)
    rcv = lax.shift_right_logical(code, 16)
    e = o + lane
    w27 = plsc.load_gather(sipack_v, [lax.shift_right_logical(e, 1)])
    c27 = lax.shift_right_logical(
        w27, (e & jnp.int32(1)) * jnp.int32(5)) & jnp.int32(31)
    q3 = lax.shift_right_logical(c27 * jnp.int32(21846), 16)
    q9 = lax.shift_right_logical(c27 * jnp.int32(7282), 16)
    si0 = c27 - jnp.int32(3) * q3
    si1 = q3 - jnp.int32(3) * q9
    si2 = q9
    sif = [si0.astype(jnp.float32), si1.astype(jnp.float32),
           si2.astype(jnp.float32)]

    snd3 = snd * jnp.int32(3)
    rcv3 = rcv * jnp.int32(3)
    ps = [plsc.load_gather(pos_v, [snd3 + jnp.int32(d)]) for d in range(3)]
    pr = [plsc.load_gather(pos_v, [rcv3 + jnp.int32(d)]) for d in range(3)]
    t = plsc.load_gather(types_v, [snd])

    d2 = _f32(1e-9)
    for d in range(3):
        v = (pr[d] - ps[d]) * _f32(10.0)
        for k in range(3):
            v = v + sif[k] * c10[k][d]
        d2 = d2 + v * v

    ibits = plsc.bitcast(d2, jnp.int32)
    y = plsc.bitcast(jnp.int32(0x5F3759DF) - lax.shift_right_logical(ibits, 1),
                     jnp.float32)
    for _ in range(3):
        y = y * (_f32(1.5) - _f32(0.5) * d2 * y * y)
    r = d2 * y
    inv_r = y

    x = r * _f32(1.0 / _RMAX)
    xc = jnp.minimum(x, _f32(1.0))

    u = _f32(_PI) * xc - _f32(_PI / 2)
    u2 = u * u
    cp = _f32(_COSC[-1])
    for cc in _COSC[-2::-1]:
        cp = cp * u2 + _f32(cc)
    sp = _f32(_SINC[-1])
    for cc in _SINC[-2::-1]:
        sp = sp * u2 + _f32(cc)
    sp = sp * u
    s1 = cp
    c1 = -sp

    t2 = _f32(2.0) * c1
    sins = [s1, t2 * s1]
    for _ in range(2, _NB):
        sins.append(t2 * sins[-1] - sins[-2])

    x3 = xc * xc * xc
    x6 = x3 * x3
    x7 = x6 * xc
    x8 = x7 * xc
    fc = _f32(1.0) - _f32(28.0) * x6 + _f32(48.0) * x7 - _f32(21.0) * x8
    w = _f32(_C_BESSEL) * fc * inv_r
    w = jnp.where(x < _f32(1.0), w, _f32(0.0))

    tl = t - jnp.int32(5) * c_idx
    in_half = (tl >= jnp.int32(0)) & (tl < jnp.int32(5))
    basew = jnp.where(in_half,
                      (rcv * jnp.int32(5) + tl) * jnp.int32(_NB),
                      jnp.int32(_TRASH))
    return basew, [s * w for s in sins]


def _sc_body(pos_hbm, types_hbm, epack_hbm, sipack_hbm, cell_hbm, zeros_hbm,
             out_hbm,
             pos_v, types_v, epack_v, sipack_v,
             idx2_v, val2_v, idx2t_v, val2t_v, cell_s, acc_v, sem):
    c_idx = lax.axis_index("c")
    s_idx = lax.axis_index("s")
    base = s_idx * _EPT

    pltpu.sync_copy(cell_hbm, cell_s)
    pltpu.sync_copy(pos_hbm, pos_v)
    pltpu.sync_copy(types_hbm, types_v)
    pltpu.sync_copy(epack_hbm.at[pl.ds(pl.multiple_of(base, 8), _EPT)],
                    epack_v)
    pltpu.sync_copy(
        sipack_hbm.at[pl.ds(pl.multiple_of(base // 2, 8), _EPT // 2)],
        sipack_v)

    pltpu.sync_copy(zeros_hbm,
                    acc_v.at[pl.ds(pl.multiple_of(s_idx * _WPT, 8), _WPT)])
    plsc.subcore_barrier()

    cvec = cell_s[...]
    c10 = [[cvec[k * 3 + d] * _f32(10.0) for d in range(3)]
           for k in range(3)]

    def batch_body(b, carry):
        o0 = b * _BATCH
        for g in range(_BATCH // 16):
            basew, rad = _edge_group(o0 + g * 16, pos_v, types_v, epack_v,
                                     sipack_v, c10, c_idx)
            for n in range(_NB):
                idx2_v[n, pl.ds(g * 16, 16)] = basew + jnp.int32(n)
                val2_v[n, pl.ds(g * 16, 16)] = rad[n]
        for n in range(_NB):
            pltpu.make_async_copy(val2_v.at[n], acc_v.at[idx2_v.at[n]],
                                  sem).start(add=True)
        for n in range(_NB):
            pltpu.make_async_copy(val2_v.at[n], acc_v.at[idx2_v.at[n]],
                                  sem).wait()
        return carry

    lax.fori_loop(0, _NFULL, batch_body, 0)

    for g in range(_TAIL // 16):
        basew, rad = _edge_group(_NFULL * _BATCH + g * 16, pos_v, types_v,
                                 epack_v, sipack_v, c10, c_idx)
        for n in range(_NB):
            p = n * _TAIL + g * 16
            idx2t_v[p // 128, pl.ds(p % 128, 16)] = basew + jnp.int32(n)
            val2t_v[p // 128, pl.ds(p % 128, 16)] = rad[n]
    for n in range(2):
        pltpu.make_async_copy(val2t_v.at[n], acc_v.at[idx2t_v.at[n]],
                              sem).start(add=True)
    for n in range(2):
        pltpu.make_async_copy(val2t_v.at[n], acc_v.at[idx2t_v.at[n]],
                              sem).wait()

    plsc.subcore_barrier()
    pltpu.sync_copy(
        acc_v.at[pl.ds(pl.multiple_of(s_idx * _WPT, 8), _WPT)],
        out_hbm.at[pl.ds(
            pl.multiple_of(c_idx * _AWORDS + s_idx * _WPT, 8), _WPT)])


_sc_call = pl.kernel(
    _sc_body,
    out_type=jax.ShapeDtypeStruct((_NC * _AWORDS,), jnp.float32),
    mesh=plsc.VectorSubcoreMesh(core_axis_name="c", subcore_axis_name="s"),
    compiler_params=pltpu.CompilerParams(needs_layout_passes=False),
    scratch_types=[
        pltpu.VMEM((_N * 3,), jnp.float32),
        pltpu.VMEM((_N,), jnp.int32),
        pltpu.VMEM((_EPT,), jnp.int32),
        pltpu.VMEM((_EPT // 2,), jnp.int32),
        pltpu.VMEM((_NB, _BATCH), jnp.int32),
        pltpu.VMEM((_NB, _BATCH), jnp.float32),
        pltpu.VMEM((2, _BATCH), jnp.int32),
        pltpu.VMEM((2, _BATCH), jnp.float32),
        pltpu.VMEM((16,), jnp.float32),
        pltpu.VMEM_SHARED((_AWORDS,), jnp.float32),
        pltpu.SemaphoreType.DMA,
    ],
)


def _tc_body(r_ref, embed_ref, wmsg_ref, wrad_ref, woutT_ref, out_ref):
    hw = jnp.dot(embed_ref[...], wmsg_ref[...],
                 preferred_element_type=jnp.float32,
                 precision=lax.Precision.HIGHEST)
    wbig = (hw[:, None, :] * wrad_ref[...][None, :, :]).reshape(
        _NT * _NB, _D)
    agg = jnp.dot(r_ref[...], wbig, preferred_element_type=jnp.float32,
                  precision=lax.Precision.HIGHEST)
    act = agg * jax.nn.sigmoid(agg)
    node_e = act * woutT_ref[...]
    total = jnp.sum(node_e) * jnp.float32(_EV_TO_KJ_MOL)
    out_ref[...] = jnp.reshape(total, (1, 1))


_tc_call = pl.pallas_call(
    _tc_body,
    out_shape=jax.ShapeDtypeStruct((1, 1), jnp.float32),
)


@jax.jit
def kernel(positions, boxVectors, edge_index, shifts_idx, node_types,
           embed, W_msg, W_rad, W_out):
    cell_flat = jnp.pad(jnp.reshape(boxVectors, (9,)), (0, 7))
    epack = edge_index[0] + edge_index[1] * jnp.int32(65536)
    c27 = (shifts_idx[:, 0] + shifts_idx[:, 1] * jnp.int32(3)
           + shifts_idx[:, 2] * jnp.int32(9)).reshape(_E // 2, 2)
    sipack = c27[:, 0] + (c27[:, 1] << 5)
    zeros = jnp.zeros((_WPT,), jnp.float32)
    pos_flat = jnp.reshape(positions, (3 * _N,))
    r2 = _sc_call(pos_flat, node_types, epack, sipack, cell_flat, zeros)
    halves = jnp.reshape(r2, (_NC, _AWORDS))[:, :_NBKT * _NB]
    rfull = jnp.concatenate(
        [jnp.reshape(halves[0], (_N, 5 * _NB)),
         jnp.reshape(halves[1], (_N, 5 * _NB))], axis=1)
    out = _tc_call(rfull, embed, W_msg, W_rad, jnp.reshape(W_out, (1, _D)))
    return out[0, 0]
